# baseline (speedup 1.0000x reference)
import jax
import jax.numpy as jnp
from jax import lax
from jax.experimental import pallas as pl
from jax.experimental.pallas import tpu as pltpu
from jax.experimental.pallas import tpu_sc as plsc

B = 16384
D = 32
NROWS = 1000001
NC = 2
NS = 16
L = 16
NW = NC * NS
BPW = B // NW
CHUNK = 128
NCHUNK = BPW // CHUNK
NGROUP = BPW // L
Y_LO, Y_HI = 1.0, 5.0


def _mf_kernel(users_hbm, items_hbm, uef_hbm, ief_hbm, ub_hbm, ib_hbm,
               out_hbm, idx_u, idx_i, eidx_u, eidx_i, uec, iec,
               ub_v, ib_v, out_v, sem):
    wid = lax.axis_index("s") * NC + lax.axis_index("c")

    pltpu.sync_copy(users_hbm.at[wid], idx_u)
    pltpu.sync_copy(items_hbm.at[wid], idx_i)

    iota = lax.iota(jnp.int32, L)
    for q in range(NCHUNK):
        for l in range(CHUNK // L):
            s = pl.ds(l * L, L)
            cu = idx_u[q, s]
            ci = idx_i[q, s]
            for d in range(D):
                off = d * NROWS
                eidx_u[d * NCHUNK + q, s] = cu + off
                eidx_i[d * NCHUNK + q, s] = ci + off

    copies = []
    for d in range(D):
        for q in range(NCHUNK):
            r = d * NCHUNK + q
            dst = pl.ds(q * CHUNK, CHUNK)
            copies.append(pltpu.make_async_copy(
                uef_hbm.at[eidx_u.at[r]], uec.at[d, dst], sem))
            copies.append(pltpu.make_async_copy(
                ief_hbm.at[eidx_i.at[r]], iec.at[d, dst], sem))
    for q in range(NCHUNK):
        dst = pl.ds(q * CHUNK, CHUNK)
        copies.append(pltpu.make_async_copy(
            ub_hbm.at[idx_u.at[q]], ub_v.at[dst], sem))
        copies.append(pltpu.make_async_copy(
            ib_hbm.at[idx_i.at[q]], ib_v.at[dst], sem))
    for c in copies:
        c.start()
    for c in copies:
        c.wait()

    def group_body(g, carry):
        base = pl.multiple_of(g * L, L)
        s = pl.ds(base, L)
        acc = ub_v[s] + ib_v[s]
        for d in range(D):
            acc = acc + uec[d, s] * iec[d, s]
        y = 1.0 / (1.0 + jnp.exp(-acc))
        out_v[s] = y * (Y_HI - Y_LO) + Y_LO
        return carry

    lax.fori_loop(0, NGROUP, group_body, 0)

    pltpu.sync_copy(out_v, out_hbm.at[pl.ds(wid * BPW, BPW)])


@jax.jit
def kernel(users, items, user_emb, item_emb, user_bias, item_bias):
    users = users.astype(jnp.int32).reshape(NW, NCHUNK, CHUNK)
    items = items.astype(jnp.int32).reshape(NW, NCHUNK, CHUNK)
    ub = user_bias.reshape(-1)
    ib = item_bias.reshape(-1)
    uef = user_emb.T.reshape(-1)
    ief = item_emb.T.reshape(-1)

    mesh = plsc.VectorSubcoreMesh(core_axis_name="c", subcore_axis_name="s")
    run = pl.kernel(
        _mf_kernel,
        out_type=jax.ShapeDtypeStruct((B,), jnp.float32),
        mesh=mesh,
        compiler_params=pltpu.CompilerParams(
            needs_layout_passes=False, use_tc_tiling_on_sc=False),
        scratch_types=[
            pltpu.VMEM((NCHUNK, CHUNK), jnp.int32),
            pltpu.VMEM((NCHUNK, CHUNK), jnp.int32),
            pltpu.VMEM((D * NCHUNK, CHUNK), jnp.int32),
            pltpu.VMEM((D * NCHUNK, CHUNK), jnp.int32),
            pltpu.VMEM((D, BPW), jnp.float32),
            pltpu.VMEM((D, BPW), jnp.float32),
            pltpu.VMEM((BPW,), jnp.float32),
            pltpu.VMEM((BPW,), jnp.float32),
            pltpu.VMEM((BPW,), jnp.float32),
            pltpu.SemaphoreType.DMA,
        ],
    )
    return run(users, items, uef, ief, ub, ib)


# speedup vs baseline: 5.7644x; 5.7644x over previous
import jax
import jax.numpy as jnp
from jax import lax
from jax.experimental import pallas as pl
from jax.experimental.pallas import tpu as pltpu
from jax.experimental.pallas import tpu_sc as plsc

B = 16384
D = 32
NC = 2
NS = 16
L = 16
NW = NC * NS
BPW = B // NW
CHUNK = 128
NCHUNK = BPW // CHUNK
NGROUP = BPW // L
Y_LO, Y_HI = 1.0, 5.0


def _mf_kernel(users_hbm, items_hbm, ue_hbm, ie_hbm, ub_hbm, ib_hbm,
               out_hbm, idx_u, idx_i, ue_v, ie_v, ub_v, ib_v, out_v, sem):
    wid = lax.axis_index("s") * NC + lax.axis_index("c")

    pltpu.sync_copy(users_hbm.at[wid], idx_u)
    pltpu.sync_copy(items_hbm.at[wid], idx_i)

    copies = []
    for j in range(NCHUNK):
        rows = pl.ds(j * CHUNK, CHUNK)
        copies.append(pltpu.make_async_copy(
            ue_hbm.at[idx_u.at[j]], ue_v.at[rows], sem))
        copies.append(pltpu.make_async_copy(
            ie_hbm.at[idx_i.at[j]], ie_v.at[rows], sem))
        copies.append(pltpu.make_async_copy(
            ub_hbm.at[idx_u.at[j]], ub_v.at[rows], sem))
        copies.append(pltpu.make_async_copy(
            ib_hbm.at[idx_i.at[j]], ib_v.at[rows], sem))
    for c in copies:
        c.start()
    for c in copies:
        c.wait()

    def acc_body(g, carry):
        base = pl.multiple_of(g * L, L)
        out_v[pl.ds(base, L)] = ub_v[pl.ds(base, L)] + ib_v[pl.ds(base, L)]
        for k in range(L):
            r = base + k
            part = (ue_v[r, pl.ds(0, L)] * ie_v[r, pl.ds(0, L)]
                    + ue_v[r, pl.ds(L, L)] * ie_v[r, pl.ds(L, L)])
            idx = jnp.full((L,), r, jnp.int32)
            plsc.addupdate_scatter(out_v, [idx], part)
        return carry

    lax.fori_loop(0, NGROUP, acc_body, 0)

    def act_body(g, carry):
        base = pl.multiple_of(g * L, L)
        acc = out_v[pl.ds(base, L)]
        y = 1.0 / (1.0 + jnp.exp(-acc))
        out_v[pl.ds(base, L)] = y * (Y_HI - Y_LO) + Y_LO
        return carry

    lax.fori_loop(0, NGROUP, act_body, 0)

    pltpu.sync_copy(out_v, out_hbm.at[pl.ds(wid * BPW, BPW)])


@jax.jit
def kernel(users, items, user_emb, item_emb, user_bias, item_bias):
    users = users.astype(jnp.int32).reshape(NW, NCHUNK, CHUNK)
    items = items.astype(jnp.int32).reshape(NW, NCHUNK, CHUNK)
    ub = user_bias.reshape(-1)
    ib = item_bias.reshape(-1)

    mesh = plsc.VectorSubcoreMesh(core_axis_name="c", subcore_axis_name="s")
    run = pl.kernel(
        _mf_kernel,
        out_type=jax.ShapeDtypeStruct((B,), jnp.float32),
        mesh=mesh,
        compiler_params=pltpu.CompilerParams(
            needs_layout_passes=False, use_tc_tiling_on_sc=False),
        scratch_types=[
            pltpu.VMEM((NCHUNK, CHUNK), jnp.int32),
            pltpu.VMEM((NCHUNK, CHUNK), jnp.int32),
            pltpu.VMEM((BPW, D), jnp.float32),
            pltpu.VMEM((BPW, D), jnp.float32),
            pltpu.VMEM((BPW,), jnp.float32),
            pltpu.VMEM((BPW,), jnp.float32),
            pltpu.VMEM((BPW,), jnp.float32),
            pltpu.SemaphoreType.DMA,
        ],
    )
    return run(users, items, user_emb, item_emb, ub, ib)
